# baseline (speedup 1.0000x reference)
import jax
import jax.numpy as jnp
from jax import lax
from jax.experimental import pallas as pl
from jax.experimental.pallas import tpu as pltpu
from jax.experimental.pallas import tpu_sc as plsc

VOCAB = 100000
HIDDEN = 768
MAX_POS = 4096
EPS = 1e-12
B, S = 4, 4096
N = B * S

NC, NS = 2, 16
NW = NC * NS
SLICES = 4
NH = N // SLICES
BH = B // SLICES
TPW = NH // NW
C = 32
NB = 4
NCHUNK = TPW // C

TB = 1024
PB = S // TB


def _gather_body(tid_hbm, word_hbm, out_hbm, idx_all, buf, gsem, osem):
    cid = lax.axis_index("c")
    sid = lax.axis_index("s")
    wid = sid * NC + cid
    base = wid * TPW

    pltpu.sync_copy(tid_hbm.at[pl.ds(base, TPW)], idx_all)

    def gather(c, b):
        return pltpu.make_async_copy(
            word_hbm.at[idx_all.at[pl.ds(c * C, C)]], buf.at[b], gsem.at[b])

    def out(c, b):
        return pltpu.make_async_copy(
            buf.at[b], out_hbm.at[pl.ds(base + c * C, C)], osem.at[b])

    gather(0, 0).start()
    gather(1, 1).start()

    def chunk_body(c, carry):
        b = lax.rem(c, NB)
        pb = lax.rem(c + 2, NB)

        @pl.when(c + 2 < NCHUNK)
        def _prefetch():
            @pl.when(c >= 2)
            def _free():
                out(c - 2, pb).wait()
            gather(c + 2, pb).start()

        gather(c, b).wait()
        out(c, b).start()
        return carry

    lax.fori_loop(0, NCHUNK, chunk_body, 0)
    for k in range(NB):
        out(NCHUNK - NB + k, lax.rem(NCHUNK - NB + k, NB)).wait()


def _ln_math(w, pos, idsf, type_ref, gamma_ref, beta_ref):
    t0 = type_ref[0, :]
    dl = type_ref[1, :] - t0
    emb = w + pos + t0[None, :] + idsf[:, None] * dl[None, :]
    mean = jnp.mean(emb, axis=-1, keepdims=True)
    cent = emb - mean
    var = jnp.mean(cent * cent, axis=-1, keepdims=True)
    normed = cent * lax.rsqrt(var + EPS)
    return normed * gamma_ref[0, :][None, :] + beta_ref[0, :][None, :]


def _ln_body(w_ref, pos_ref, ids_ref, type_ref, gamma_ref, beta_ref, o_ref):
    o_ref[...] = _ln_math(w_ref[...], pos_ref[...],
                          ids_ref[0, 0, :].astype(jnp.float32),
                          type_ref, gamma_ref, beta_ref)


def _ln_body_carry(w_ref, pos_ref, ids_ref, type_ref, gamma_ref, beta_ref,
                   carry_ref, o_ref):
    del carry_ref
    o_ref[...] = _ln_math(w_ref[...], pos_ref[...],
                          ids_ref[0, 0, :].astype(jnp.float32),
                          type_ref, gamma_ref, beta_ref)


def kernel(token_ids, token_type_ids, word_emb, pos_emb, type_emb, gamma, beta):
    tid = token_ids.reshape(N).astype(jnp.int32)
    tt3 = token_type_ids.reshape(N // TB, 1, TB).astype(jnp.int32)
    gamma2 = gamma.reshape(1, HIDDEN)
    beta2 = beta.reshape(1, HIDDEN)

    mesh = plsc.VectorSubcoreMesh(core_axis_name="c", subcore_axis_name="s",
                                  num_cores=NC, num_subcores=NS)
    sc_gather = pl.kernel(
        _gather_body,
        out_type=jax.ShapeDtypeStruct((NH, HIDDEN), jnp.float32),
        mesh=mesh,
        compiler_params=pltpu.CompilerParams(needs_layout_passes=False),
        scratch_types=[
            pltpu.VMEM((TPW,), jnp.int32),
            pltpu.VMEM((NB, C, HIDDEN), jnp.float32),
            pltpu.SemaphoreType.DMA((NB,)),
            pltpu.SemaphoreType.DMA((NB,)),
        ],
    )

    def make_specs(s):
        return [
            pl.BlockSpec((TB, HIDDEN), lambda i, b: (b * PB + i, 0)),
            pl.BlockSpec((TB, HIDDEN), lambda i, b: (i, 0)),
            pl.BlockSpec((1, 1, TB),
                         lambda i, b: ((s * BH + b) * PB + i, 0, 0)),
            pl.BlockSpec((2, HIDDEN), lambda i, b: (0, 0)),
            pl.BlockSpec((1, HIDDEN), lambda i, b: (0, 0)),
            pl.BlockSpec((1, HIDDEN), lambda i, b: (0, 0)),
        ]

    def out_spec(s):
        return pl.BlockSpec((TB, HIDDEN),
                            lambda i, b: ((s * BH + b) * PB + i, 0))

    gathered = [sc_gather(tid[h * NH:(h + 1) * NH], word_emb)
                for h in range(SLICES)]

    acc = pl.pallas_call(
        _ln_body,
        out_shape=jax.ShapeDtypeStruct((N, HIDDEN), jnp.float32),
        grid=(PB, BH),
        in_specs=make_specs(0),
        out_specs=out_spec(0),
    )(gathered[0], pos_emb, tt3, type_emb, gamma2, beta2)

    for s in range(1, SLICES):
        acc = pl.pallas_call(
            _ln_body_carry,
            out_shape=jax.ShapeDtypeStruct((N, HIDDEN), jnp.float32),
            grid=(PB, BH),
            in_specs=make_specs(s) + [
                pl.BlockSpec((8, HIDDEN), lambda i, b: (0, 0)),
            ],
            out_specs=out_spec(s),
            input_output_aliases={6: 0},
        )(gathered[s], pos_emb, tt3, type_emb, gamma2, beta2, acc)

    return acc.reshape(B, S, HIDDEN)


# speedup vs baseline: 1.1388x; 1.1388x over previous
import jax
import jax.numpy as jnp
from jax import lax
from jax.experimental import pallas as pl
from jax.experimental.pallas import tpu as pltpu
from jax.experimental.pallas import tpu_sc as plsc

VOCAB = 100000
HIDDEN = 768
MAX_POS = 4096
EPS = 1e-12
B, S = 4, 4096
N = B * S

NC, NS = 2, 16
NW = NC * NS
SLICES = 2
NH = N // SLICES
SH = S // SLICES
TPW = NH // NW
C = 32
NB = 4
NCHUNK = TPW // C

TB = SH


def _gather_body(tid_hbm, word_hbm, out_hbm, idx_all, buf, gsem, osem):
    cid = lax.axis_index("c")
    sid = lax.axis_index("s")
    wid = sid * NC + cid
    base = wid * TPW

    pltpu.sync_copy(tid_hbm.at[pl.ds(base, TPW)], idx_all)

    def gather(c, b):
        return pltpu.make_async_copy(
            word_hbm.at[idx_all.at[pl.ds(c * C, C)]], buf.at[b], gsem.at[b])

    def out(c, b):
        return pltpu.make_async_copy(
            buf.at[b], out_hbm.at[pl.ds(base + c * C, C)], osem.at[b])

    gather(0, 0).start()
    gather(1, 1).start()

    def chunk_body(c, carry):
        b = lax.rem(c, NB)
        pb = lax.rem(c + 2, NB)

        @pl.when(c + 2 < NCHUNK)
        def _prefetch():
            @pl.when(c >= 2)
            def _free():
                out(c - 2, pb).wait()
            gather(c + 2, pb).start()

        gather(c, b).wait()
        out(c, b).start()
        return carry

    lax.fori_loop(0, NCHUNK, chunk_body, 0)
    for k in range(NB):
        out(NCHUNK - NB + k, lax.rem(NCHUNK - NB + k, NB)).wait()


def _ln_math(w, pos, idsf, type_ref, gamma_ref, beta_ref):
    t0 = type_ref[0, :]
    dl = type_ref[1, :] - t0
    emb = w + pos + t0[None, :] + idsf[:, None] * dl[None, :]
    mean = jnp.mean(emb, axis=-1, keepdims=True)
    cent = emb - mean
    var = jnp.mean(cent * cent, axis=-1, keepdims=True)
    normed = cent * lax.rsqrt(var + EPS)
    return normed * gamma_ref[0, :][None, :] + beta_ref[0, :][None, :]


def _ln_body(w_ref, pos_ref, ids_ref, type_ref, gamma_ref, beta_ref, o_ref):
    o_ref[...] = _ln_math(w_ref[...], pos_ref[...],
                          ids_ref[0, 0, :].astype(jnp.float32),
                          type_ref, gamma_ref, beta_ref)


def _ln_body_carry(w_ref, pos_ref, ids_ref, type_ref, gamma_ref, beta_ref,
                   carry_ref, o_ref):
    del carry_ref
    o_ref[...] = _ln_math(w_ref[...], pos_ref[...],
                          ids_ref[0, 0, :].astype(jnp.float32),
                          type_ref, gamma_ref, beta_ref)


def kernel(token_ids, token_type_ids, word_emb, pos_emb, type_emb, gamma, beta):
    tt3 = token_type_ids.reshape(N // TB, 1, TB).astype(jnp.int32)
    gamma2 = gamma.reshape(1, HIDDEN)
    beta2 = beta.reshape(1, HIDDEN)

    mesh = plsc.VectorSubcoreMesh(core_axis_name="c", subcore_axis_name="s",
                                  num_cores=NC, num_subcores=NS)
    sc_gather = pl.kernel(
        _gather_body,
        out_type=jax.ShapeDtypeStruct((NH, HIDDEN), jnp.float32),
        mesh=mesh,
        compiler_params=pltpu.CompilerParams(needs_layout_passes=False),
        scratch_types=[
            pltpu.VMEM((TPW,), jnp.int32),
            pltpu.VMEM((NB, C, HIDDEN), jnp.float32),
            pltpu.SemaphoreType.DMA((NB,)),
            pltpu.SemaphoreType.DMA((NB,)),
        ],
    )

    def make_specs(s):
        return [
            pl.BlockSpec((TB, HIDDEN), lambda b: (b, 0)),
            pl.BlockSpec((TB, HIDDEN), lambda b: (s, 0)),
            pl.BlockSpec((1, 1, TB), lambda b: (b * SLICES + s, 0, 0)),
            pl.BlockSpec((2, HIDDEN), lambda b: (0, 0)),
            pl.BlockSpec((1, HIDDEN), lambda b: (0, 0)),
            pl.BlockSpec((1, HIDDEN), lambda b: (0, 0)),
        ]

    def out_spec(s):
        return pl.BlockSpec((TB, HIDDEN), lambda b: (b * SLICES + s, 0))

    gathered = [
        sc_gather(token_ids[:, s * SH:(s + 1) * SH].reshape(NH)
                  .astype(jnp.int32), word_emb)
        for s in range(SLICES)
    ]

    acc = pl.pallas_call(
        _ln_body,
        out_shape=jax.ShapeDtypeStruct((N, HIDDEN), jnp.float32),
        grid=(B,),
        in_specs=make_specs(0),
        out_specs=out_spec(0),
    )(gathered[0], pos_emb, tt3, type_emb, gamma2, beta2)

    for s in range(1, SLICES):
        acc = pl.pallas_call(
            _ln_body_carry,
            out_shape=jax.ShapeDtypeStruct((N, HIDDEN), jnp.float32),
            grid=(B,),
            in_specs=make_specs(s) + [
                pl.BlockSpec((8, HIDDEN), lambda b: (0, 0)),
            ],
            out_specs=out_spec(s),
            input_output_aliases={6: 0},
        )(gathered[s], pos_emb, tt3, type_emb, gamma2, beta2, acc)

    return acc.reshape(B, S, HIDDEN)


# speedup vs baseline: 1.1391x; 1.0002x over previous
import jax
import jax.numpy as jnp
from jax import lax
from jax.experimental import pallas as pl
from jax.experimental.pallas import tpu as pltpu
from jax.experimental.pallas import tpu_sc as plsc

VOCAB = 100000
HIDDEN = 768
MAX_POS = 4096
EPS = 1e-12
B, S = 4, 4096
N = B * S

NC, NS = 2, 16
NW = NC * NS
SLICES = 2
NH = N // SLICES
SH = S // SLICES
TPW = NH // NW
C = 32
NB = 4
NCHUNK = TPW // C

TB = SH


def _gather_body(tid_hbm, word_hbm, out_hbm, idx_all, buf, gsem, osem):
    cid = lax.axis_index("c")
    sid = lax.axis_index("s")
    wid = sid * NC + cid
    base = wid * TPW

    pltpu.sync_copy(tid_hbm.at[pl.ds(base, TPW)], idx_all)

    def gather(c, b):
        return pltpu.make_async_copy(
            word_hbm.at[idx_all.at[pl.ds(c * C, C)]], buf.at[b], gsem.at[b])

    def out(c, b):
        return pltpu.make_async_copy(
            buf.at[b], out_hbm.at[pl.ds(base + c * C, C)], osem.at[b])

    gather(0, 0).start()
    gather(1, 1).start()

    def chunk_body(c, carry):
        b = lax.rem(c, NB)
        pb = lax.rem(c + 2, NB)

        @pl.when(c + 2 < NCHUNK)
        def _prefetch():
            @pl.when(c + 2 >= NB)
            def _free():
                out(c + 2 - NB, pb).wait()
            gather(c + 2, pb).start()

        gather(c, b).wait()
        out(c, b).start()
        return carry

    lax.fori_loop(0, NCHUNK, chunk_body, 0)
    for k in range(NB):
        cc = NCHUNK - NB + k
        out(cc, lax.rem(cc, NB)).wait()


def _ln_math(w, pos, idsf, type_ref, gamma_ref, beta_ref):
    t0 = type_ref[0, :]
    dl = type_ref[1, :] - t0
    emb = w + pos + t0[None, :] + idsf[:, None] * dl[None, :]
    mean = jnp.mean(emb, axis=-1, keepdims=True)
    cent = emb - mean
    var = jnp.mean(cent * cent, axis=-1, keepdims=True)
    normed = cent * lax.rsqrt(var + EPS)
    return normed * gamma_ref[0, :][None, :] + beta_ref[0, :][None, :]


def _ln_body(w_ref, pos_ref, ids_ref, type_ref, gamma_ref, beta_ref, o_ref):
    o_ref[...] = _ln_math(w_ref[...], pos_ref[...],
                          ids_ref[0, 0, :].astype(jnp.float32),
                          type_ref, gamma_ref, beta_ref)


def _ln_body_carry(w_ref, pos_ref, ids_ref, type_ref, gamma_ref, beta_ref,
                   carry_ref, o_ref):
    del carry_ref
    o_ref[...] = _ln_math(w_ref[...], pos_ref[...],
                          ids_ref[0, 0, :].astype(jnp.float32),
                          type_ref, gamma_ref, beta_ref)


def kernel(token_ids, token_type_ids, word_emb, pos_emb, type_emb, gamma, beta):
    tt3 = token_type_ids.reshape(N // TB, 1, TB).astype(jnp.int32)
    gamma2 = gamma.reshape(1, HIDDEN)
    beta2 = beta.reshape(1, HIDDEN)

    mesh = plsc.VectorSubcoreMesh(core_axis_name="c", subcore_axis_name="s",
                                  num_cores=NC, num_subcores=NS)
    sc_gather = pl.kernel(
        _gather_body,
        out_type=jax.ShapeDtypeStruct((NH, HIDDEN), jnp.float32),
        mesh=mesh,
        compiler_params=pltpu.CompilerParams(needs_layout_passes=False),
        scratch_types=[
            pltpu.VMEM((TPW,), jnp.int32),
            pltpu.VMEM((NB, C, HIDDEN), jnp.float32),
            pltpu.SemaphoreType.DMA((NB,)),
            pltpu.SemaphoreType.DMA((NB,)),
        ],
    )

    def make_specs(s):
        return [
            pl.BlockSpec((TB, HIDDEN), lambda b: (b, 0)),
            pl.BlockSpec((TB, HIDDEN), lambda b: (s, 0)),
            pl.BlockSpec((1, 1, TB), lambda b: (b * SLICES + s, 0, 0)),
            pl.BlockSpec((2, HIDDEN), lambda b: (0, 0)),
            pl.BlockSpec((1, HIDDEN), lambda b: (0, 0)),
            pl.BlockSpec((1, HIDDEN), lambda b: (0, 0)),
        ]

    def out_spec(s):
        return pl.BlockSpec((TB, HIDDEN), lambda b: (b * SLICES + s, 0))

    gathered = [
        sc_gather(token_ids[:, s * SH:(s + 1) * SH].reshape(NH)
                  .astype(jnp.int32), word_emb)
        for s in range(SLICES)
    ]

    acc = pl.pallas_call(
        _ln_body,
        out_shape=jax.ShapeDtypeStruct((N, HIDDEN), jnp.float32),
        grid=(B,),
        in_specs=make_specs(0),
        out_specs=out_spec(0),
    )(gathered[0], pos_emb, tt3, type_emb, gamma2, beta2)

    for s in range(1, SLICES):
        acc = pl.pallas_call(
            _ln_body_carry,
            out_shape=jax.ShapeDtypeStruct((N, HIDDEN), jnp.float32),
            grid=(B,),
            in_specs=make_specs(s) + [
                pl.BlockSpec((8, HIDDEN), lambda b: (0, 0)),
            ],
            out_specs=out_spec(s),
            input_output_aliases={6: 0},
        )(gathered[s], pos_emb, tt3, type_emb, gamma2, beta2, acc)

    return acc.reshape(B, S, HIDDEN)
